# baseline (speedup 1.0000x reference)
import functools

import jax
import jax.numpy as jnp
from jax import lax
from jax.experimental import pallas as pl
from jax.experimental.pallas import tpu as pltpu
from jax.experimental.pallas import tpu_sc as plsc

H = 2048
W = 2048
NC = 2
NS = 16
L = 16
NW = NC * NS
ROWS_PER_W = H // NW
CH = 16
NCHUNK = ROWS_PER_W // CH
ROW_SLICES = W // L
UNROLL = 4

_mesh = plsc.VectorSubcoreMesh(
    core_axis_name="c", subcore_axis_name="s", num_cores=NC, num_subcores=NS
)


@functools.partial(
    pl.kernel,
    out_type=jax.ShapeDtypeStruct((NW, 4, L), jnp.float32),
    mesh=_mesh,
    scratch_types=[
        pltpu.VMEM((CH, W), jnp.float32),
        pltpu.VMEM((4, L), jnp.float32),
    ],
)
def _partials(r_hbm, o_hbm, out_hbm, buf, acc_v):
    wid = lax.axis_index("s") * NC + lax.axis_index("c")
    row_base = wid * ROWS_PER_W

    def reduce_one(src, acc_slot):
        nacc0 = jnp.zeros((L,), jnp.float32)
        wacc0 = jnp.zeros((L,), jnp.float32)

        def chunk_body(g, carry):
            nacc, wacc = carry
            row0 = row_base + g * CH
            pltpu.sync_copy(src.at[pl.ds(row0, CH), :], buf)

            def row_body(r, carry):
                nacc, wacc = carry

                def sl_body(j, accs):
                    accs = list(accs)
                    for u in range(UNROLL):
                        x = buf[r, pl.ds((j * UNROLL + u) * L, L)]
                        accs[u] = accs[u] + jnp.where(x != 0.0, 1.0, 0.0)
                    return tuple(accs)

                accs = lax.fori_loop(
                    0,
                    ROW_SLICES // UNROLL,
                    sl_body,
                    tuple(jnp.zeros((L,), jnp.float32) for _ in range(UNROLL)),
                )
                rsum = accs[0] + accs[1] + accs[2] + accs[3]
                cw = (row0 + r - (H // 2)).astype(jnp.float32)
                return nacc + rsum, wacc + cw * rsum

            return lax.fori_loop(0, CH, row_body, (nacc, wacc))

        nacc, wacc = lax.fori_loop(0, NCHUNK, chunk_body, (nacc0, wacc0))
        acc_v[2 * acc_slot] = nacc
        acc_v[2 * acc_slot + 1] = wacc

    reduce_one(r_hbm, 0)
    reduce_one(o_hbm, 1)
    pltpu.sync_copy(acc_v, out_hbm.at[wid])


def kernel(rendered_silhouette, original_silhouette):
    p = _partials(rendered_silhouette, original_silhouette)
    s = jnp.sum(p.astype(jnp.int32), axis=(0, 2))
    nr = s[0].astype(jnp.float32)
    wr = s[1].astype(jnp.float32)
    no = s[2].astype(jnp.float32)
    wo = s[3].astype(jnp.float32)
    d = wr / nr - wo / no
    return (d * d) * (2.0 / (H * W))


# speedup vs baseline: 1.2349x; 1.2349x over previous
import functools

import jax
import jax.numpy as jnp
from jax import lax
from jax.experimental import pallas as pl
from jax.experimental.pallas import tpu as pltpu
from jax.experimental.pallas import tpu_sc as plsc

H = 2048
W = 2048
NC = 2
NS = 16
L = 16
NW = NC * NS
ROWS_PER_W = H // NW
CH = 16
NCHUNK = ROWS_PER_W // CH
ROW_SLICES = W // L
UNROLL = 8

_mesh = plsc.VectorSubcoreMesh(
    core_axis_name="c", subcore_axis_name="s", num_cores=NC, num_subcores=NS
)


@functools.partial(
    pl.kernel,
    out_type=jax.ShapeDtypeStruct((NW, 4, L), jnp.float32),
    mesh=_mesh,
    scratch_types=[
        pltpu.VMEM((CH, W), jnp.float32),
        pltpu.VMEM((CH, W), jnp.float32),
        pltpu.VMEM((4, L), jnp.float32),
        pltpu.SemaphoreType.DMA,
        pltpu.SemaphoreType.DMA,
    ],
)
def _partials(r_hbm, o_hbm, out_hbm, buf0, buf1, acc_v, sem0, sem1):
    wid = lax.axis_index("s") * NC + lax.axis_index("c")
    row_base = wid * ROWS_PER_W
    bufs = (buf0, buf1)
    sems = (sem0, sem1)

    chunks = [(r_hbm, g * CH) for g in range(NCHUNK)]
    chunks += [(o_hbm, g * CH) for g in range(NCHUNK)]

    def start(g):
        src, off = chunks[g]
        return pltpu.async_copy(
            src.at[pl.ds(row_base + off, CH), :], bufs[g % 2], sems[g % 2]
        )

    def reduce_chunk(g, nacc, wacc):
        buf = bufs[g % 2]
        row0 = row_base + chunks[g][1]

        def row_body(r, carry):
            nacc, wacc = carry

            def sl_body(j, accs):
                accs = list(accs)
                for u in range(UNROLL):
                    x = buf[r, pl.ds((j * UNROLL + u) * L, L)]
                    accs[u] = accs[u] + jnp.where(x != 0.0, 1.0, 0.0)
                return tuple(accs)

            accs = lax.fori_loop(
                0,
                ROW_SLICES // UNROLL,
                sl_body,
                tuple(jnp.zeros((L,), jnp.float32) for _ in range(UNROLL)),
            )
            rsum = accs[0]
            for u in range(1, UNROLL):
                rsum = rsum + accs[u]
            cw = (row0 + r - (H // 2)).astype(jnp.float32)
            return nacc + rsum, wacc + cw * rsum

        return lax.fori_loop(0, CH, row_body, (nacc, wacc))

    zero = jnp.zeros((L,), jnp.float32)
    totals = [zero, zero, zero, zero]
    descs = {0: start(0), 1: start(1)}
    for g in range(2 * NCHUNK):
        descs.pop(g).wait()
        ai = g // NCHUNK
        n, w = reduce_chunk(g, totals[2 * ai], totals[2 * ai + 1])
        totals[2 * ai], totals[2 * ai + 1] = n, w
        if g + 2 < 2 * NCHUNK:
            descs[g + 2] = start(g + 2)

    acc_v[0] = totals[0]
    acc_v[1] = totals[1]
    acc_v[2] = totals[2]
    acc_v[3] = totals[3]
    pltpu.sync_copy(acc_v, out_hbm.at[wid])


def kernel(rendered_silhouette, original_silhouette):
    p = _partials(rendered_silhouette, original_silhouette)
    s = jnp.sum(p.astype(jnp.int32), axis=(0, 2))
    nr = s[0].astype(jnp.float32)
    wr = s[1].astype(jnp.float32)
    no = s[2].astype(jnp.float32)
    wo = s[3].astype(jnp.float32)
    d = wr / nr - wo / no
    return (d * d) * (2.0 / (H * W))
